# Initial kernel scaffold; baseline (speedup 1.0000x reference)
import jax
import jax.numpy as jnp
from jax.experimental import pallas as pl


def kernel(context_ids, pos_ids, neg_ids, in_embed_weight, out_embed_weight):
    raise NotImplementedError("write your pallas kernel here")


# speedup vs baseline: 5.2340x; 5.2340x over previous
import functools

import jax
import jax.numpy as jnp
from jax import lax
from jax.experimental import pallas as pl
from jax.experimental.pallas import tpu as pltpu
from jax.experimental.pallas import tpu_sc as plsc

B = 16384
D = 64
CTX = 20
NEG = 20
V = 1000000

NC = 2
NS = 16
NW = NC * NS
LANES = 16

BPW = B // NW
CB = 32
NCHUNK = BPW // CB
GROUPS = CB * CTX // 128
SCOL = 32


def _sc_scores(ctx_idx, pos_ids, neg_idx, in_w, out_w):
  mesh = plsc.VectorSubcoreMesh(core_axis_name="c", subcore_axis_name="s")

  @functools.partial(
      pl.kernel,
      mesh=mesh,
      out_type=jax.ShapeDtypeStruct((B, SCOL), jnp.float32),
      compiler_params=pltpu.CompilerParams(use_tc_tiling_on_sc=False),
      scratch_types=[
          pltpu.VMEM((CB * CTX,), jnp.int32),
          pltpu.VMEM((CB * NEG,), jnp.int32),
          pltpu.VMEM((CB,), jnp.int32),
          pltpu.VMEM((CB * CTX, D), jnp.float32),
          pltpu.VMEM((CB * NEG, D), jnp.float32),
          pltpu.VMEM((CB, D), jnp.float32),
          pltpu.VMEM((CB, SCOL), jnp.float32),
          pltpu.SemaphoreType.DMA,
      ],
  )
  def k(ctx_idx_hbm, pos_hbm, neg_idx_hbm, in_hbm, out_hbm, scores_hbm,
        ctx_idx_v, neg_idx_v, pos_idx_v, ctx_rows, neg_rows, pos_rows,
        scores_v, sem):
    wid = lax.axis_index("s") * NC + lax.axis_index("c")

    def chunk_body(chunk, _):
      b0 = wid * BPW + chunk * CB
      pltpu.sync_copy(ctx_idx_hbm.at[pl.ds(b0 * CTX, CB * CTX)], ctx_idx_v)
      pltpu.sync_copy(neg_idx_hbm.at[pl.ds(b0 * NEG, CB * NEG)], neg_idx_v)
      pltpu.sync_copy(pos_hbm.at[pl.ds(b0, CB)], pos_idx_v)
      copies = []
      for g in range(GROUPS):
        copies.append(pltpu.async_copy(
            in_hbm.at[ctx_idx_v.at[pl.ds(g * 128, 128)]],
            ctx_rows.at[pl.ds(g * 128, 128)], sem))
      for g in range(GROUPS):
        copies.append(pltpu.async_copy(
            out_hbm.at[neg_idx_v.at[pl.ds(g * 128, 128)]],
            neg_rows.at[pl.ds(g * 128, 128)], sem))
      copies.append(pltpu.async_copy(out_hbm.at[pos_idx_v], pos_rows, sem))
      for cp in copies:
        cp.wait()

      lane = lax.iota(jnp.int32, LANES)
      rot = [(lane + sh) & (LANES - 1) for sh in (8, 4, 2, 1)]

      dnums = lax.GatherDimensionNumbers(
          offset_dims=(), collapsed_slice_dims=(0,), start_index_map=(0,))

      def hsum(v):
        for r in rot:
          v = v + lax.gather(
              v, r[:, None], dimension_numbers=dnums, slice_sizes=(1,),
              mode=lax.GatherScatterMode.PROMISE_IN_BOUNDS)
        return v

      def batch_body(b, _):
        base = b * CTX
        mean = [ctx_rows[base, pl.ds(q * LANES, LANES)] for q in range(4)]
        for c in range(1, CTX):
          for q in range(4):
            mean[q] = mean[q] + ctx_rows[base + c, pl.ds(q * LANES, LANES)]
        scale = jnp.float32(1.0 / CTX)
        mean = [m * scale for m in mean]

        def dot_row(row_ref, r):
          p = mean[0] * row_ref[r, pl.ds(0, LANES)]
          for q in range(1, 4):
            p = p + mean[q] * row_ref[r, pl.ds(q * LANES, LANES)]
          return hsum(p)

        s_lo = jnp.zeros((LANES,), jnp.float32)
        s_hi = jnp.zeros((LANES,), jnp.float32)
        s_lo = jnp.where(lane == 0, dot_row(pos_rows, b), s_lo)
        for n in range(NEG):
          j = 1 + n
          s = dot_row(neg_rows, b * NEG + n)
          if j < LANES:
            s_lo = jnp.where(lane == j, s, s_lo)
          else:
            s_hi = jnp.where(lane == j - LANES, s, s_hi)
        scores_v[b, pl.ds(0, LANES)] = s_lo
        scores_v[b, pl.ds(LANES, LANES)] = s_hi
        return 0

      lax.fori_loop(0, CB, batch_body, 0)
      pltpu.sync_copy(scores_v, scores_hbm.at[pl.ds(b0, CB)])
      return 0

    lax.fori_loop(0, NCHUNK, chunk_body, 0)

  return k(ctx_idx, pos_ids, neg_idx, in_w, out_w)


def _tc_loss(scores):
  rows = B * SCOL // 128

  def body(s_ref, o_ref):
    s = s_ref[...]
    col = lax.broadcasted_iota(jnp.int32, s.shape, 1) % SCOL
    def sp(x):
      return jnp.maximum(x, 0.0) + jnp.log(1.0 + jnp.exp(-jnp.abs(x)))
    contrib = jnp.where(col == 0, sp(-s), 0.0)
    contrib = contrib + jnp.where((col >= 1) & (col <= NEG), sp(s), 0.0)
    o_ref[0, 0] = jnp.sum(contrib) * jnp.float32(1.0 / B)

  out = pl.pallas_call(
      body,
      out_shape=jax.ShapeDtypeStruct((1, 1), jnp.float32),
      in_specs=[pl.BlockSpec((rows, 128), lambda: (0, 0))],
      out_specs=pl.BlockSpec((1, 1), lambda: (0, 0),
                             memory_space=pltpu.SMEM),
  )(scores.reshape(rows, 128))
  return out[0, 0]


def kernel(context_ids, pos_ids, neg_ids, in_embed_weight, out_embed_weight):
  ctx_idx = context_ids.reshape(B * CTX)
  neg_idx = neg_ids.reshape(B * NEG)
  scores = _sc_scores(ctx_idx, pos_ids, neg_idx,
                      in_embed_weight, out_embed_weight)
  return _tc_loss(scores)
